# baseline (speedup 1.0000x reference)
import jax
import jax.numpy as jnp
from jax.experimental import pallas as pl
from jax.experimental.pallas import tpu as pltpu

_B, _S, _D = 2, 2048, 2048
_NB = 64
_K = 8
_FF = 2048
_TAU = 10.0
_EPS = 1e-6
_LN_EPS = 1e-5

_TM = 256
_FC = 512


def _fused_kernel(x_ref, lnw_ref, lnb_ref, wr_ref, br_ref, u_ref, v_ref,
                  gamma_ref, w1_ref, w2_ref, out_ref):
    x = x_ref[...]
    xb = x.astype(jnp.bfloat16)
    inv_d = 1.0 / x.shape[1]

    static = None
    for f in range(_FF // _FC):
        w1c = w1_ref[pl.ds(f * _FC, _FC), :]
        h1 = jax.lax.dot_general(xb, w1c, (((1,), (1,)), ((), ())),
                                 preferred_element_type=jnp.float32)
        h1 = 0.5 * h1 * (1.0 + jax.lax.erf(h1 * 0.7071067811865476))
        w2c = w2_ref[:, pl.ds(f * _FC, _FC)]
        part = jax.lax.dot_general(h1.astype(jnp.bfloat16), w2c,
                                   (((1,), (1,)), ((), ())),
                                   preferred_element_type=jnp.float32)
        static = part if static is None else static + part

    mu = jnp.sum(x, axis=-1, keepdims=True) * inv_d
    ex2 = jnp.sum(x * x, axis=-1, keepdims=True) * inv_d
    rs = jax.lax.rsqrt(ex2 - mu * mu + _LN_EPS)

    wr = wr_ref[...]
    a_mat = wr * lnw_ref[...]
    c1 = jnp.sum(a_mat, axis=-1, keepdims=True).reshape(1, _NB)
    c2 = (jnp.sum(wr * lnb_ref[...], axis=-1, keepdims=True).reshape(1, _NB)
          + br_ref[...])
    xa = jax.lax.dot_general(xb, a_mat.astype(jnp.bfloat16),
                             (((1,), (1,)), ((), ())),
                             preferred_element_type=jnp.float32)
    r = rs * (xa - mu * c1) + c2
    r = jnp.clip(r, -_TAU, _TAU)
    alpha = jax.nn.softplus(r)

    iota = jax.lax.broadcasted_iota(jnp.int32, alpha.shape, 1)
    ab = jax.lax.bitcast_convert_type(alpha, jnp.int32)
    key = jax.lax.bitwise_or(jax.lax.bitwise_and(ab, -64), 63 - iota)
    sel = jnp.zeros(alpha.shape, jnp.bool_)
    work = key
    for _ in range(_K):
        m = jnp.max(work, axis=-1, keepdims=True)
        pick = work == m
        sel = jnp.logical_or(sel, pick)
        work = jnp.where(pick, jnp.int32(-2147483648), work)

    phi = jnp.where(sel, alpha, 0.0)
    s_sum = jnp.sum(phi, axis=-1, keepdims=True)
    z = phi * (jnp.tanh(s_sum) / (s_sum + _EPS))

    u_raw = u_ref[...]
    v_raw = v_ref[...]
    u_n = jnp.maximum(jnp.sqrt(jnp.sum(u_raw * u_raw, axis=-1, keepdims=True)), _EPS)
    v_n = jnp.maximum(jnp.sqrt(jnp.sum(v_raw * v_raw, axis=-1, keepdims=True)), _EPS)
    u_norm = (u_raw / u_n).astype(jnp.bfloat16)
    v_eff = ((v_raw / v_n) * gamma_ref[...]).astype(jnp.bfloat16)

    h_lat = jax.lax.dot_general(xb, u_norm, (((1,), (1,)), ((), ())),
                                preferred_element_type=jnp.float32)
    hw = (h_lat * z).astype(jnp.bfloat16)
    dyn = jax.lax.dot_general(hw, v_eff, (((1,), (0,)), ((), ())),
                              preferred_element_type=jnp.float32)

    out_ref[...] = static + dyn


@jax.jit
def kernel(x, ln_w, ln_b, Wr, br, raw_U, raw_V, gamma, W1, W2):
    Bv, Sv, Dv = x.shape
    n = Bv * Sv
    x_flat = x.reshape(n, Dv)
    grid = (n // _TM,)

    full = lambda a: pl.BlockSpec(a.shape, lambda i: (0,) * a.ndim)
    out = pl.pallas_call(
        _fused_kernel,
        grid=grid,
        in_specs=[
            pl.BlockSpec((_TM, Dv), lambda i: (i, 0)),
            full(ln_w.reshape(1, Dv)),
            full(ln_b.reshape(1, Dv)),
            full(Wr),
            full(br.reshape(1, _NB)),
            full(raw_U),
            full(raw_V),
            full(gamma.reshape(1, Dv)),
            full(W1),
            full(W2),
        ],
        out_specs=pl.BlockSpec((_TM, Dv), lambda i: (i, 0)),
        out_shape=jax.ShapeDtypeStruct((n, Dv), jnp.float32),
        compiler_params=pltpu.CompilerParams(
            dimension_semantics=("parallel",),
        ),
    )(x_flat, ln_w.reshape(1, Dv), ln_b.reshape(1, Dv), Wr,
      br.reshape(1, _NB), raw_U, raw_V, gamma.reshape(1, Dv), W1, W2)
    return out.reshape(Bv, Sv, Dv)


# speedup vs baseline: 1.1586x; 1.1586x over previous
import jax
import jax.numpy as jnp
from jax.experimental import pallas as pl
from jax.experimental.pallas import tpu as pltpu

_B, _S, _D = 2, 2048, 2048
_NB = 64
_K = 8
_FF = 2048
_TAU = 10.0
_EPS = 1e-6
_LN_EPS = 1e-5

_TM = 512
_FC = 1024


def _fused_kernel(x_ref, lnw_ref, lnb_ref, wr_ref, br_ref, u_ref, v_ref,
                  gamma_ref, w1_ref, w2_ref, out_ref):
    x = x_ref[...]
    xb = x.astype(jnp.bfloat16)
    inv_d = 1.0 / x.shape[1]

    static = None
    for f in range(_FF // _FC):
        w1c = w1_ref[pl.ds(f * _FC, _FC), :]
        h1 = jax.lax.dot_general(xb, w1c, (((1,), (1,)), ((), ())),
                                 preferred_element_type=jnp.float32)
        h1 = 0.5 * h1 * (1.0 + jax.lax.erf(h1 * 0.7071067811865476))
        w2c = w2_ref[:, pl.ds(f * _FC, _FC)]
        part = jax.lax.dot_general(h1.astype(jnp.bfloat16), w2c,
                                   (((1,), (1,)), ((), ())),
                                   preferred_element_type=jnp.float32)
        static = part if static is None else static + part

    mu = jnp.sum(x, axis=-1, keepdims=True) * inv_d
    ex2 = jnp.sum(x * x, axis=-1, keepdims=True) * inv_d
    rs = jax.lax.rsqrt(ex2 - mu * mu + _LN_EPS)

    wr = wr_ref[...]
    a_mat = wr * lnw_ref[...]
    c1 = jnp.sum(a_mat, axis=-1, keepdims=True).reshape(1, _NB)
    c2 = (jnp.sum(wr * lnb_ref[...], axis=-1, keepdims=True).reshape(1, _NB)
          + br_ref[...])
    xa = jax.lax.dot_general(xb, a_mat.astype(jnp.bfloat16),
                             (((1,), (1,)), ((), ())),
                             preferred_element_type=jnp.float32)
    r = rs * (xa - mu * c1) + c2
    r = jnp.clip(r, -_TAU, _TAU)
    alpha = jax.nn.softplus(r)

    iota = jax.lax.broadcasted_iota(jnp.int32, alpha.shape, 1)
    ab = jax.lax.bitcast_convert_type(alpha, jnp.int32)
    key = jax.lax.bitwise_or(jax.lax.bitwise_and(ab, -64), 63 - iota)
    sel = jnp.zeros(alpha.shape, jnp.bool_)
    work = key
    for _ in range(_K):
        m = jnp.max(work, axis=-1, keepdims=True)
        pick = work == m
        sel = jnp.logical_or(sel, pick)
        work = jnp.where(pick, jnp.int32(-2147483648), work)

    phi = jnp.where(sel, alpha, 0.0)
    s_sum = jnp.sum(phi, axis=-1, keepdims=True)
    z = phi * (jnp.tanh(s_sum) / (s_sum + _EPS))

    u_raw = u_ref[...]
    v_raw = v_ref[...]
    u_n = jnp.maximum(jnp.sqrt(jnp.sum(u_raw * u_raw, axis=-1, keepdims=True)), _EPS)
    v_n = jnp.maximum(jnp.sqrt(jnp.sum(v_raw * v_raw, axis=-1, keepdims=True)), _EPS)
    u_norm = (u_raw / u_n).astype(jnp.bfloat16)
    v_eff = ((v_raw / v_n) * gamma_ref[...]).astype(jnp.bfloat16)

    h_lat = jax.lax.dot_general(xb, u_norm, (((1,), (1,)), ((), ())),
                                preferred_element_type=jnp.float32)
    hw = (h_lat * z).astype(jnp.bfloat16)
    dyn = jax.lax.dot_general(hw, v_eff, (((1,), (0,)), ((), ())),
                              preferred_element_type=jnp.float32)

    out_ref[...] = static + dyn


@jax.jit
def kernel(x, ln_w, ln_b, Wr, br, raw_U, raw_V, gamma, W1, W2):
    Bv, Sv, Dv = x.shape
    n = Bv * Sv
    x_flat = x.reshape(n, Dv)
    grid = (n // _TM,)

    full = lambda a: pl.BlockSpec(a.shape, lambda i: (0,) * a.ndim)
    out = pl.pallas_call(
        _fused_kernel,
        grid=grid,
        in_specs=[
            pl.BlockSpec((_TM, Dv), lambda i: (i, 0)),
            full(ln_w.reshape(1, Dv)),
            full(ln_b.reshape(1, Dv)),
            full(Wr),
            full(br.reshape(1, _NB)),
            full(raw_U),
            full(raw_V),
            full(gamma.reshape(1, Dv)),
            full(W1),
            full(W2),
        ],
        out_specs=pl.BlockSpec((_TM, Dv), lambda i: (i, 0)),
        out_shape=jax.ShapeDtypeStruct((n, Dv), jnp.float32),
        compiler_params=pltpu.CompilerParams(
            dimension_semantics=("parallel",),
        ),
    )(x_flat, ln_w.reshape(1, Dv), ln_b.reshape(1, Dv), Wr,
      br.reshape(1, _NB), raw_U, raw_V, gamma.reshape(1, Dv), W1, W2)
    return out.reshape(Bv, Sv, Dv)


# speedup vs baseline: 1.1729x; 1.0124x over previous
import jax
import jax.numpy as jnp
from jax.experimental import pallas as pl
from jax.experimental.pallas import tpu as pltpu

_B, _S, _D = 2, 2048, 2048
_NB = 64
_K = 8
_FF = 2048
_TAU = 10.0
_EPS = 1e-6
_LN_EPS = 1e-5

_TM = 512
_FC = 2048


def _fused_kernel(x_ref, lnw_ref, lnb_ref, wr_ref, br_ref, u_ref, v_ref,
                  gamma_ref, w1_ref, w2_ref, out_ref):
    x = x_ref[...]
    xb = x.astype(jnp.bfloat16)
    inv_d = 1.0 / x.shape[1]

    static = None
    for f in range(_FF // _FC):
        w1c = w1_ref[pl.ds(f * _FC, _FC), :]
        h1 = jax.lax.dot_general(xb, w1c, (((1,), (1,)), ((), ())),
                                 preferred_element_type=jnp.float32)
        h1 = 0.5 * h1 * (1.0 + jax.lax.erf(h1 * 0.7071067811865476))
        w2c = w2_ref[:, pl.ds(f * _FC, _FC)]
        part = jax.lax.dot_general(h1.astype(jnp.bfloat16), w2c,
                                   (((1,), (1,)), ((), ())),
                                   preferred_element_type=jnp.float32)
        static = part if static is None else static + part

    mu = jnp.sum(x, axis=-1, keepdims=True) * inv_d
    ex2 = jnp.sum(x * x, axis=-1, keepdims=True) * inv_d
    rs = jax.lax.rsqrt(ex2 - mu * mu + _LN_EPS)

    wr = wr_ref[...]
    a_mat = wr * lnw_ref[...]
    c1 = jnp.sum(a_mat, axis=-1, keepdims=True).reshape(1, _NB)
    c2 = (jnp.sum(wr * lnb_ref[...], axis=-1, keepdims=True).reshape(1, _NB)
          + br_ref[...])
    xa = jax.lax.dot_general(xb, a_mat.astype(jnp.bfloat16),
                             (((1,), (1,)), ((), ())),
                             preferred_element_type=jnp.float32)
    r = rs * (xa - mu * c1) + c2
    r = jnp.clip(r, -_TAU, _TAU)
    alpha = jax.nn.softplus(r)

    iota = jax.lax.broadcasted_iota(jnp.int32, alpha.shape, 1)
    ab = jax.lax.bitcast_convert_type(alpha, jnp.int32)
    key = jax.lax.bitwise_or(jax.lax.bitwise_and(ab, -64), 63 - iota)
    sel = jnp.zeros(alpha.shape, jnp.bool_)
    work = key
    for _ in range(_K):
        m = jnp.max(work, axis=-1, keepdims=True)
        pick = work == m
        sel = jnp.logical_or(sel, pick)
        work = jnp.where(pick, jnp.int32(-2147483648), work)

    phi = jnp.where(sel, alpha, 0.0)
    s_sum = jnp.sum(phi, axis=-1, keepdims=True)
    z = phi * (jnp.tanh(s_sum) / (s_sum + _EPS))

    u_raw = u_ref[...]
    v_raw = v_ref[...]
    u_n = jnp.maximum(jnp.sqrt(jnp.sum(u_raw * u_raw, axis=-1, keepdims=True)), _EPS)
    v_n = jnp.maximum(jnp.sqrt(jnp.sum(v_raw * v_raw, axis=-1, keepdims=True)), _EPS)
    u_norm = (u_raw / u_n).astype(jnp.bfloat16)
    v_eff = ((v_raw / v_n) * gamma_ref[...]).astype(jnp.bfloat16)

    h_lat = jax.lax.dot_general(xb, u_norm, (((1,), (1,)), ((), ())),
                                preferred_element_type=jnp.float32)
    hw = (h_lat * z).astype(jnp.bfloat16)
    dyn = jax.lax.dot_general(hw, v_eff, (((1,), (0,)), ((), ())),
                              preferred_element_type=jnp.float32)

    out_ref[...] = static + dyn


@jax.jit
def kernel(x, ln_w, ln_b, Wr, br, raw_U, raw_V, gamma, W1, W2):
    Bv, Sv, Dv = x.shape
    n = Bv * Sv
    x_flat = x.reshape(n, Dv)
    grid = (n // _TM,)

    full = lambda a: pl.BlockSpec(a.shape, lambda i: (0,) * a.ndim)
    out = pl.pallas_call(
        _fused_kernel,
        grid=grid,
        in_specs=[
            pl.BlockSpec((_TM, Dv), lambda i: (i, 0)),
            full(ln_w.reshape(1, Dv)),
            full(ln_b.reshape(1, Dv)),
            full(Wr),
            full(br.reshape(1, _NB)),
            full(raw_U),
            full(raw_V),
            full(gamma.reshape(1, Dv)),
            full(W1),
            full(W2),
        ],
        out_specs=pl.BlockSpec((_TM, Dv), lambda i: (i, 0)),
        out_shape=jax.ShapeDtypeStruct((n, Dv), jnp.float32),
        compiler_params=pltpu.CompilerParams(
            dimension_semantics=("parallel",),
        ),
    )(x_flat, ln_w.reshape(1, Dv), ln_b.reshape(1, Dv), Wr,
      br.reshape(1, _NB), raw_U, raw_V, gamma.reshape(1, Dv), W1, W2)
    return out.reshape(Bv, Sv, Dv)


# speedup vs baseline: 1.1739x; 1.0008x over previous
import jax
import jax.numpy as jnp
from jax.experimental import pallas as pl
from jax.experimental.pallas import tpu as pltpu

_B, _S, _D = 2, 2048, 2048
_NB = 64
_K = 8
_FF = 2048
_TAU = 10.0
_EPS = 1e-6
_LN_EPS = 1e-5

_TM = 512
_FC = 2048


def _fused_kernel(x_ref, lnw_ref, lnb_ref, wr_ref, br_ref, u_ref, v_ref,
                  gamma_ref, w1_ref, w2_ref, out_ref):
    x = x_ref[...]
    xb = x.astype(jnp.bfloat16)
    inv_d = 1.0 / x.shape[1]

    static = None
    for f in range(_FF // _FC):
        w1c = w1_ref[pl.ds(f * _FC, _FC), :]
        h1 = jax.lax.dot_general(xb, w1c, (((1,), (1,)), ((), ())),
                                 preferred_element_type=jnp.float32)
        h1 = 0.5 * h1 * (1.0 + jax.lax.erf(h1 * 0.7071067811865476))
        w2c = w2_ref[:, pl.ds(f * _FC, _FC)]
        part = jax.lax.dot_general(h1.astype(jnp.bfloat16), w2c,
                                   (((1,), (1,)), ((), ())),
                                   preferred_element_type=jnp.float32)
        static = part if static is None else static + part

    mu = jnp.sum(x, axis=-1, keepdims=True) * inv_d
    ex2 = jnp.sum(x * x, axis=-1, keepdims=True) * inv_d
    rs = jax.lax.rsqrt(ex2 - mu * mu + _LN_EPS)

    wr = wr_ref[...]
    a_mat = wr * lnw_ref[...]
    c1 = jnp.sum(a_mat, axis=-1, keepdims=True).reshape(1, _NB)
    c2 = (jnp.sum(wr * lnb_ref[...], axis=-1, keepdims=True).reshape(1, _NB)
          + br_ref[...])
    xa = jax.lax.dot_general(xb, a_mat.astype(jnp.bfloat16),
                             (((1,), (1,)), ((), ())),
                             preferred_element_type=jnp.float32)
    r = rs * (xa - mu * c1) + c2
    r = jnp.clip(r, -_TAU, _TAU)
    alpha = jax.nn.softplus(r)

    iota = jax.lax.broadcasted_iota(jnp.int32, alpha.shape, 1)
    ab = jax.lax.bitcast_convert_type(alpha, jnp.int32)
    key = jax.lax.bitwise_or(jax.lax.bitwise_and(ab, -64), 63 - iota)
    sel = jnp.zeros(alpha.shape, jnp.bool_)
    work = key
    for _ in range(_K):
        m = jnp.max(work, axis=-1, keepdims=True)
        pick = work == m
        sel = jnp.logical_or(sel, pick)
        work = jnp.where(pick, jnp.int32(-2147483648), work)

    phi = jnp.where(sel, alpha, 0.0)
    s_sum = jnp.sum(phi, axis=-1, keepdims=True)
    z = phi * (jnp.tanh(s_sum) / (s_sum + _EPS))

    u_raw = u_ref[...]
    v_raw = v_ref[...]
    u_n = jnp.maximum(jnp.sqrt(jnp.sum(u_raw * u_raw, axis=-1, keepdims=True)), _EPS)
    v_n = jnp.maximum(jnp.sqrt(jnp.sum(v_raw * v_raw, axis=-1, keepdims=True)), _EPS)
    u_norm = (u_raw / u_n).astype(jnp.bfloat16)
    v_eff = ((v_raw / v_n) * gamma_ref[...]).astype(jnp.bfloat16)

    h_lat = jax.lax.dot_general(xb, u_norm, (((1,), (1,)), ((), ())),
                                preferred_element_type=jnp.float32)
    hw = (h_lat * z).astype(jnp.bfloat16)
    dyn = jax.lax.dot_general(hw, v_eff, (((1,), (0,)), ((), ())),
                              preferred_element_type=jnp.float32)

    out_ref[...] = static + dyn


@jax.jit
def kernel(x, ln_w, ln_b, Wr, br, raw_U, raw_V, gamma, W1, W2):
    Bv, Sv, Dv = x.shape
    n = Bv * Sv
    x_flat = x.reshape(n, Dv)
    grid = (n // _TM,)

    full = lambda a: pl.BlockSpec(a.shape, lambda i: (0,) * a.ndim)
    out = pl.pallas_call(
        _fused_kernel,
        grid=grid,
        in_specs=[
            pl.BlockSpec((_TM, Dv), lambda i: (i, 0)),
            full(ln_w.reshape(1, Dv)),
            full(ln_b.reshape(1, Dv)),
            full(Wr),
            full(br.reshape(1, _NB)),
            full(raw_U),
            full(raw_V),
            full(gamma.reshape(1, Dv)),
            full(W1),
            full(W2),
        ],
        out_specs=pl.BlockSpec((_TM, Dv), lambda i: (i, 0)),
        out_shape=jax.ShapeDtypeStruct((n, Dv), jnp.float32),
        compiler_params=pltpu.CompilerParams(
            dimension_semantics=("arbitrary",),
        ),
    )(x_flat, ln_w.reshape(1, Dv), ln_b.reshape(1, Dv), Wr,
      br.reshape(1, _NB), raw_U, raw_V, gamma.reshape(1, Dv), W1, W2)
    return out.reshape(Bv, Sv, Dv)
